# Initial kernel scaffold; baseline (speedup 1.0000x reference)
import jax
import jax.numpy as jnp
from jax.experimental import pallas as pl


def kernel(x, knn, W, gamma, beta):
    raise NotImplementedError("write your pallas kernel here")


# speedup vs baseline: 16.4318x; 16.4318x over previous
import functools

import jax
import jax.numpy as jnp
from jax import lax
from jax.experimental import pallas as pl
from jax.experimental.pallas import tpu as pltpu
from jax.experimental.pallas import tpu_sc as plsc

NB = 2
NN = 10000
KK = 16
CI = 128
CO = 128
RR = NB * NN
EPS = 1e-5

NC = 2
NS = 16
NW = NC * NS
RPW = RR // NW
CHUNK = 5
NCHUNK = RPW // CHUNK
GROWS = CHUNK * KK
LANES = 16

MM_BLK = 1000
MM_GRID = RR // MM_BLK


def _mm_body(x_ref, w_ref, o_ref):
    o_ref[...] = lax.dot_general(
        x_ref[...], w_ref[...],
        (((1,), (1,)), ((), ())),
        preferred_element_type=jnp.float32)


def _matmul(x2d, w):
    return pl.pallas_call(
        _mm_body,
        grid=(MM_GRID,),
        in_specs=[
            pl.BlockSpec((MM_BLK, CI), lambda i: (i, 0)),
            pl.BlockSpec((CO, CI), lambda i: (0, 0)),
        ],
        out_specs=pl.BlockSpec((MM_BLK, CO), lambda i: (i, 0)),
        out_shape=jax.ShapeDtypeStruct((RR, CO), jnp.float32),
    )(x2d, w)


def _gm_body(h_hbm, knn_hbm, out_hbm, stats_hbm,
             idx_v, gbuf0, gbuf1, out_v, ssum_v, ssq_v, sem0, sem1):
    cid = lax.axis_index("c")
    sid = lax.axis_index("s")
    wid = sid * NC + cid
    base = wid * RPW

    pltpu.sync_copy(knn_hbm.at[pl.ds(base * KK, RPW * KK)], idx_v)

    off = jnp.full((LANES,), jnp.where(base >= NN, NN, 0), dtype=jnp.int32)

    def _off_body(i, _):
        sl = pl.ds(i * LANES, LANES)
        idx_v[sl] = idx_v[sl] + off
        return 0
    lax.fori_loop(0, RPW * KK // LANES, _off_body, 0)

    zero = jnp.zeros((LANES,), jnp.float32)
    for r in range(CO // LANES):
        ssum_v[pl.ds(r * LANES, LANES)] = zero
        ssq_v[pl.ds(r * LANES, LANES)] = zero

    def _copy(g, buf, sem):
        return pltpu.make_async_copy(
            h_hbm.at[idx_v.at[pl.ds(g * GROWS, GROWS)]], buf, sem)

    def _accum(g, buf):
        for d in range(CHUNK):
            row = g * CHUNK + d
            for r in range(CO // LANES):
                sl = pl.ds(r * LANES, LANES)
                a = buf[d * KK, sl]
                for j in range(1, KK):
                    a = a + buf[d * KK + j, sl]
                out_v[pl.ds(row * CO + r * LANES, LANES)] = a
                plsc.addupdate(ssum_v.at[sl], a)
                plsc.addupdate(ssq_v.at[sl], a * a)

    _copy(0, gbuf0, sem0).start()
    _copy(1, gbuf1, sem1).start()

    def _body(i, _):
        g0 = 2 * i
        g1 = 2 * i + 1
        _copy(g0, gbuf0, sem0).wait()
        _accum(g0, gbuf0)
        _copy(g0 + 2, gbuf0, sem0).start()

        _copy(g1, gbuf1, sem1).wait()
        _accum(g1, gbuf1)

        @pl.when(g1 + 2 < NCHUNK)
        def _():
            _copy(g1 + 2, gbuf1, sem1).start()
        return 0

    lax.fori_loop(0, (NCHUNK - 1) // 2, _body, 0)

    g_last = NCHUNK - 1
    _copy(g_last, gbuf0, sem0).wait()
    _accum(g_last, gbuf0)

    pltpu.sync_copy(out_v, out_hbm.at[pl.ds(base * CO, RPW * CO)])
    pltpu.sync_copy(ssum_v, stats_hbm.at[pl.ds(wid * CO, CO)])
    pltpu.sync_copy(ssq_v, stats_hbm.at[pl.ds((NW + wid) * CO, CO)])


@functools.lru_cache(maxsize=None)
def _make_gather_mean():
    mesh = plsc.VectorSubcoreMesh(
        core_axis_name="c", subcore_axis_name="s",
        num_cores=NC, num_subcores=NS)
    return pl.kernel(
        _gm_body,
        out_type=(
            jax.ShapeDtypeStruct((RR * CO,), jnp.float32),
            jax.ShapeDtypeStruct((2 * NW * CO,), jnp.float32),
        ),
        mesh=mesh,
        scratch_types=[
            pltpu.VMEM((RPW * KK,), jnp.int32),
            pltpu.VMEM((GROWS, CO), jnp.float32),
            pltpu.VMEM((GROWS, CO), jnp.float32),
            pltpu.VMEM((RPW * CO,), jnp.float32),
            pltpu.VMEM((CO,), jnp.float32),
            pltpu.VMEM((CO,), jnp.float32),
            pltpu.SemaphoreType.DMA,
            pltpu.SemaphoreType.DMA,
        ],
    )


def _bn_body(s_ref, st_ref, g_ref, b_ref, o_ref):
    st = st_ref[...]
    s1 = jnp.sum(st[:NW], axis=0, keepdims=True)
    s2 = jnp.sum(st[NW:], axis=0, keepdims=True)
    mean = s1 / RR
    var = s2 / RR - mean * mean
    alpha = g_ref[...] * lax.rsqrt(var + (KK * KK) * EPS)
    shift = b_ref[...] - mean * alpha
    o_ref[...] = s_ref[...] * alpha + shift


def _bn(sums, stats2d, gamma2d, beta2d):
    return pl.pallas_call(
        _bn_body,
        grid=(MM_GRID,),
        in_specs=[
            pl.BlockSpec((MM_BLK, CO), lambda i: (i, 0)),
            pl.BlockSpec((2 * NW, CO), lambda i: (0, 0)),
            pl.BlockSpec((1, CO), lambda i: (0, 0)),
            pl.BlockSpec((1, CO), lambda i: (0, 0)),
        ],
        out_specs=pl.BlockSpec((MM_BLK, CO), lambda i: (i, 0)),
        out_shape=jax.ShapeDtypeStruct((RR, CO), jnp.float32),
    )(sums, stats2d, gamma2d, beta2d)


@jax.jit
def kernel(x, knn, W, gamma, beta):
    h = _matmul(x.reshape(RR, CI), W)
    sums, stats = _make_gather_mean()(h, knn.reshape(RR * KK))
    out = _bn(sums.reshape(RR, CO), stats.reshape(2 * NW, CO),
              gamma.reshape(1, CO), beta.reshape(1, CO))
    return out.reshape(NB, NN, CO)


# speedup vs baseline: 19.9254x; 1.2126x over previous
import functools

import jax
import jax.numpy as jnp
from jax import lax
from jax.experimental import pallas as pl
from jax.experimental.pallas import tpu as pltpu
from jax.experimental.pallas import tpu_sc as plsc

NB = 2
NN = 10000
KK = 16
CI = 128
CO = 128
RR = NB * NN
EPS = 1e-5

NC = 2
NS = 16
NW = NC * NS
RPW = RR // NW
CHUNK = 5
NCHUNK = RPW // CHUNK
GROWS = CHUNK * KK
LANES = 16

MM_BLK = 1000
MM_GRID = RR // MM_BLK


def _mm_body(x_ref, w_ref, o_ref):
    o_ref[...] = lax.dot_general(
        x_ref[...], w_ref[...],
        (((1,), (1,)), ((), ())),
        preferred_element_type=jnp.float32)


def _matmul(x2d, w):
    return pl.pallas_call(
        _mm_body,
        grid=(MM_GRID,),
        in_specs=[
            pl.BlockSpec((MM_BLK, CI), lambda i: (i, 0)),
            pl.BlockSpec((CO, CI), lambda i: (0, 0)),
        ],
        out_specs=pl.BlockSpec((MM_BLK, CO), lambda i: (i, 0)),
        out_shape=jax.ShapeDtypeStruct((RR, CO), jnp.float32),
    )(x2d, w)


def _gm_body(h_hbm, knn_hbm, out_hbm, stats_hbm,
             idx_v, gbuf0, gbuf1, out_v, ssum_v, ssq_v, sem0, sem1):
    cid = lax.axis_index("c")
    sid = lax.axis_index("s")
    wid = sid * NC + cid
    base = wid * RPW

    pltpu.sync_copy(knn_hbm.at[pl.ds(base * KK, RPW * KK)], idx_v)

    off = jnp.full((LANES,), jnp.where(base >= NN, NN, 0), dtype=jnp.int32)

    def _off_body(i, _):
        sl = pl.ds(i * LANES, LANES)
        idx_v[sl] = idx_v[sl] + off
        return 0
    lax.fori_loop(0, RPW * KK // LANES, _off_body, 0)

    zero = jnp.zeros((LANES,), jnp.float32)
    for r in range(CO // LANES):
        ssum_v[pl.ds(r * LANES, LANES)] = zero
        ssq_v[pl.ds(r * LANES, LANES)] = zero

    def _copy(g, buf, sem):
        return pltpu.make_async_copy(
            h_hbm.at[idx_v.at[pl.ds(g * GROWS, GROWS)]], buf, sem)

    def _accum(g, buf):
        for d in range(CHUNK):
            row = g * CHUNK + d
            for r in range(CO // LANES):
                sl = pl.ds(r * LANES, LANES)
                v = [buf[d * KK + j, sl] for j in range(KK)]
                while len(v) > 1:
                    v = [v[2 * t] + v[2 * t + 1] for t in range(len(v) // 2)]
                a = v[0]
                out_v[pl.ds(row * CO + r * LANES, LANES)] = a
                plsc.addupdate(ssum_v.at[sl], a)
                plsc.addupdate(ssq_v.at[sl], a * a)

    _copy(0, gbuf0, sem0).start()
    _copy(1, gbuf1, sem1).start()

    def _body(i, _):
        g0 = 2 * i
        g1 = 2 * i + 1
        _copy(g0, gbuf0, sem0).wait()
        _accum(g0, gbuf0)
        _copy(g0 + 2, gbuf0, sem0).start()

        _copy(g1, gbuf1, sem1).wait()
        _accum(g1, gbuf1)

        @pl.when(g1 + 2 < NCHUNK)
        def _():
            _copy(g1 + 2, gbuf1, sem1).start()
        return 0

    lax.fori_loop(0, (NCHUNK - 1) // 2, _body, 0)

    g_last = NCHUNK - 1
    _copy(g_last, gbuf0, sem0).wait()
    _accum(g_last, gbuf0)

    pltpu.sync_copy(out_v, out_hbm.at[pl.ds(base * CO, RPW * CO)])
    pltpu.sync_copy(ssum_v, stats_hbm.at[pl.ds(wid * CO, CO)])
    pltpu.sync_copy(ssq_v, stats_hbm.at[pl.ds((NW + wid) * CO, CO)])


@functools.lru_cache(maxsize=None)
def _make_gather_mean():
    mesh = plsc.VectorSubcoreMesh(
        core_axis_name="c", subcore_axis_name="s",
        num_cores=NC, num_subcores=NS)
    return pl.kernel(
        _gm_body,
        out_type=(
            jax.ShapeDtypeStruct((RR * CO,), jnp.float32),
            jax.ShapeDtypeStruct((2 * NW * CO,), jnp.float32),
        ),
        mesh=mesh,
        scratch_types=[
            pltpu.VMEM((RPW * KK,), jnp.int32),
            pltpu.VMEM((GROWS, CO), jnp.float32),
            pltpu.VMEM((GROWS, CO), jnp.float32),
            pltpu.VMEM((RPW * CO,), jnp.float32),
            pltpu.VMEM((CO,), jnp.float32),
            pltpu.VMEM((CO,), jnp.float32),
            pltpu.SemaphoreType.DMA,
            pltpu.SemaphoreType.DMA,
        ],
    )


def _bn_body(s_ref, st_ref, g_ref, b_ref, o_ref):
    st = st_ref[...]
    s1 = jnp.sum(st[:NW], axis=0, keepdims=True)
    s2 = jnp.sum(st[NW:], axis=0, keepdims=True)
    mean = s1 / RR
    var = s2 / RR - mean * mean
    alpha = g_ref[...] * lax.rsqrt(var + (KK * KK) * EPS)
    shift = b_ref[...] - mean * alpha
    o_ref[...] = s_ref[...] * alpha + shift


def _bn(sums, stats2d, gamma2d, beta2d):
    return pl.pallas_call(
        _bn_body,
        grid=(MM_GRID,),
        in_specs=[
            pl.BlockSpec((MM_BLK, CO), lambda i: (i, 0)),
            pl.BlockSpec((2 * NW, CO), lambda i: (0, 0)),
            pl.BlockSpec((1, CO), lambda i: (0, 0)),
            pl.BlockSpec((1, CO), lambda i: (0, 0)),
        ],
        out_specs=pl.BlockSpec((MM_BLK, CO), lambda i: (i, 0)),
        out_shape=jax.ShapeDtypeStruct((RR, CO), jnp.float32),
    )(sums, stats2d, gamma2d, beta2d)


@jax.jit
def kernel(x, knn, W, gamma, beta):
    h = _matmul(x.reshape(RR, CI), W)
    sums, stats = _make_gather_mean()(h, knn.reshape(RR * KK))
    out = _bn(sums.reshape(RR, CO), stats.reshape(2 * NW, CO),
              gamma.reshape(1, CO), beta.reshape(1, CO))
    return out.reshape(NB, NN, CO)
